# baseline (speedup 1.0000x reference)
import functools

import jax
import jax.numpy as jnp
from jax import lax
from jax.experimental import pallas as pl
from jax.experimental.pallas import tpu as pltpu
from jax.experimental.pallas import tpu_sc as plsc

N_NODES = 10000
D = 128
NC = 2
NS = 16
NW = NC * NS
K = 64
SLOW_FRAC = 0.369
NBUF = 2
N_PAD = 10240
ROWS_PER_TILE = N_PAD // NS


def _sc_aggregate(x_pad, src, dst, n0, n1, with_deg):
    mesh = plsc.VectorSubcoreMesh(core_axis_name="c", subcore_axis_name="s")

    out_type = [jax.ShapeDtypeStruct((NC, N_PAD, D), jnp.float32)]
    scratch = (
        [pltpu.VMEM((K,), jnp.int32) for _ in range(2 * NBUF)]
        + [pltpu.VMEM((K, D), jnp.float32) for _ in range(NBUF)]
        + [pltpu.VMEM_SHARED((N_PAD, D), jnp.float32)]
        + [pltpu.SemaphoreType.DMA for _ in range(2 * NBUF)]
    )
    if with_deg:
        out_type.append(jax.ShapeDtypeStruct((NC, N_PAD), jnp.float32))
        scratch += [
            pltpu.VMEM((K,), jnp.float32),
            pltpu.VMEM((K,), jnp.float32),
            pltpu.VMEM_SHARED((N_PAD,), jnp.float32),
        ]

    @functools.partial(pl.kernel, mesh=mesh, out_type=out_type,
                       scratch_types=scratch)
    def agg(x_hbm, src_hbm, dst_hbm, *refs):
        n_out = 2 if with_deg else 1
        acc_hbm = refs[0]
        p = n_out
        sv = refs[p:p + NBUF]; p += NBUF
        dv = refs[p:p + NBUF]; p += NBUF
        rv = refs[p:p + NBUF]; p += NBUF
        acc_sh = refs[p]; p += 1
        semi = refs[p:p + NBUF]; p += NBUF
        semr = refs[p:p + NBUF]; p += NBUF
        if with_deg:
            deg_hbm = refs[1]
            ones_v, zdeg_v, deg_sh = refs[p:p + 3]
        c = lax.axis_index("c")
        s = lax.axis_index("s")
        n_chunks = jnp.where(c == 0, n0, n1)
        base = jnp.where(c == 0, s * (n0 * K),
                         NS * (n0 * K) + s * (n1 * K))

        zero16 = jnp.zeros((16,), jnp.float32)

        def fill_zrow(i, _):
            r = i // (D // 16)
            col = (i % (D // 16)) * 16
            rv[0][r, pl.ds(col, 16)] = zero16
            return 0
        lax.fori_loop(0, K * (D // 16), fill_zrow, 0)

        if with_deg:
            one16 = jnp.ones((16,), jnp.float32)
            def fill_deg_bufs(i, _):
                zdeg_v[pl.ds(i * 16, 16)] = zero16
                ones_v[pl.ds(i * 16, 16)] = one16
                return 0
            lax.fori_loop(0, K // 16, fill_deg_bufs, 0)

        row0 = s * ROWS_PER_TILE
        def zero_acc(i, _):
            pltpu.sync_copy(rv[0], acc_sh.at[pl.ds(row0 + i * K, K)])
            if with_deg:
                pltpu.sync_copy(zdeg_v,
                                deg_sh.at[pl.ds(row0 + i * K, K)])
            return 0
        lax.fori_loop(0, ROWS_PER_TILE // K, zero_acc, 0)

        plsc.subcore_barrier()

        def idx_load(g, j):
            pltpu.async_copy(src_hbm.at[pl.ds(base + g * K, K)],
                             sv[j], semi[j])
            pltpu.async_copy(dst_hbm.at[pl.ds(base + g * K, K)],
                             dv[j], semi[j])

        def idx_wait(g, j):
            pltpu.make_async_copy(
                src_hbm.at[pl.ds(base + g * K, K)], sv[j], semi[j]).wait()
            pltpu.make_async_copy(
                dst_hbm.at[pl.ds(base + g * K, K)], dv[j], semi[j]).wait()

        def g_start(j):
            pltpu.async_copy(x_hbm.at[sv[j]], rv[j], semr[j])

        def g_wait(j):
            pltpu.make_async_copy(x_hbm.at[sv[j]], rv[j], semr[j]).wait()

        def scat(j):
            pltpu.sync_copy(rv[j], acc_sh.at[dv[j]], add=True)
            if with_deg:
                pltpu.sync_copy(ones_v, deg_sh.at[dv[j]], add=True)

        for j in range(NBUF):
            idx_load(j, j)
        for j in range(NBUF - 1):
            idx_wait(j, j)
            g_start(j)

        def body(t, _):
            g0 = t * NBUF
            for j in range(NBUF):
                g = g0 + j
                jp = (j + NBUF - 1) % NBUF
                idx_wait(g + NBUF - 1, jp)
                g_start(jp)
                g_wait(j)
                scat(j)
                idx_load(g + NBUF, j)
            return 0
        lax.fori_loop(0, n_chunks // NBUF - 1, body, 0)

        idx_wait(n_chunks - 1, NBUF - 1)
        g_start(NBUF - 1)
        for j in range(NBUF):
            g_wait(j)
            scat(j)

        plsc.subcore_barrier()

        pltpu.sync_copy(acc_sh.at[pl.ds(row0, ROWS_PER_TILE)],
                        acc_hbm.at[c, pl.ds(row0, ROWS_PER_TILE)])
        if with_deg:
            pltpu.sync_copy(deg_sh.at[pl.ds(row0, ROWS_PER_TILE)],
                            deg_hbm.at[c, pl.ds(row0, ROWS_PER_TILE)])

    res = agg(x_pad, src, dst)
    if not isinstance(res, (list, tuple)):
        res = (res,)
    return tuple(res)


BLK = 1000


def _tc_layer(acc, deg, x, wn, ws, b2d, relu):
    def body(acc_ref, deg_ref, x_ref, wn_ref, ws_ref, b_ref, o_ref):
        a = acc_ref[0] + acc_ref[1]
        dg = deg_ref[0] + deg_ref[1]
        mean = a / jnp.maximum(dg, 1.0)
        out = jnp.dot(mean, wn_ref[...], preferred_element_type=jnp.float32)
        out = out + jnp.dot(x_ref[...], ws_ref[...],
                            preferred_element_type=jnp.float32)
        out = out + b_ref[...]
        if relu:
            out = jnp.maximum(out, 0.0)
        o_ref[...] = out

    return pl.pallas_call(
        body,
        grid=(N_NODES // BLK,),
        in_specs=[
            pl.BlockSpec((NC, BLK, D), lambda i: (0, i, 0)),
            pl.BlockSpec((NC, BLK, 1), lambda i: (0, i, 0)),
            pl.BlockSpec((BLK, D), lambda i: (i, 0)),
            pl.BlockSpec((D, D), lambda i: (0, 0)),
            pl.BlockSpec((D, D), lambda i: (0, 0)),
            pl.BlockSpec((1, D), lambda i: (0, 0)),
        ],
        out_specs=pl.BlockSpec((BLK, D), lambda i: (i, 0)),
        out_shape=jax.ShapeDtypeStruct((N_NODES, D), jnp.float32),
    )(acc, deg, x, wn, ws, b2d)


def kernel(x, edge_index, W_neigh1, W_self1, b1, W_neigh2, W_self2, b2):
    src = edge_index[0].astype(jnp.int32)
    dst = edge_index[1].astype(jnp.int32)
    e = src.shape[0]
    tp = 2 * (((e + NBUF * NW * K - 1) // (NBUF * NW * K)) * NBUF)
    n0 = max(2 * NBUF, int(round(tp * SLOW_FRAC / 2)) * 2)
    n1 = tp - n0
    e_pad = NS * K * (n0 + n1)
    if e_pad > e:
        src = jnp.concatenate([src, jnp.zeros((e_pad - e,), jnp.int32)])
        dst = jnp.concatenate([dst, jnp.full((e_pad - e,), N_NODES, jnp.int32)])

    acc1, deg = _sc_aggregate(x, src, dst, n0, n1, with_deg=True)
    deg = deg.reshape(NC, N_PAD, 1)
    h = _tc_layer(acc1, deg, x, W_neigh1, W_self1,
                  b1.reshape(1, D), relu=True)
    (acc2,) = _sc_aggregate(h, src, dst, n0, n1, with_deg=False)
    return _tc_layer(acc2, deg, h, W_neigh2, W_self2,
                     b2.reshape(1, D), relu=False)


# speedup vs baseline: 1.2375x; 1.2375x over previous
import functools

import jax
import jax.numpy as jnp
from jax import lax
from jax.experimental import pallas as pl
from jax.experimental.pallas import tpu as pltpu
from jax.experimental.pallas import tpu_sc as plsc

N_NODES = 10000
D = 128
NC = 2
NS = 16
NW = NC * NS
K = 64
SLOW_FRAC = 0.631
NBUF = 2
N_PAD = 10240
ROWS_PER_TILE = N_PAD // NS


def _sc_aggregate(x_pad, src, dst, n0, n1, with_deg):
    mesh = plsc.VectorSubcoreMesh(core_axis_name="c", subcore_axis_name="s")

    out_type = [jax.ShapeDtypeStruct((NC, N_PAD, D), jnp.float32)]
    scratch = (
        [pltpu.VMEM((K,), jnp.int32) for _ in range(2 * NBUF)]
        + [pltpu.VMEM((K, D), jnp.float32) for _ in range(NBUF)]
        + [pltpu.VMEM_SHARED((N_PAD, D), jnp.float32)]
        + [pltpu.SemaphoreType.DMA for _ in range(2 * NBUF)]
    )
    if with_deg:
        out_type.append(jax.ShapeDtypeStruct((NC, N_PAD), jnp.float32))
        scratch += [
            pltpu.VMEM((K,), jnp.float32),
            pltpu.VMEM((K,), jnp.float32),
            pltpu.VMEM_SHARED((N_PAD,), jnp.float32),
        ]

    @functools.partial(pl.kernel, mesh=mesh, out_type=out_type,
                       scratch_types=scratch)
    def agg(x_hbm, src_hbm, dst_hbm, *refs):
        n_out = 2 if with_deg else 1
        acc_hbm = refs[0]
        p = n_out
        sv = refs[p:p + NBUF]; p += NBUF
        dv = refs[p:p + NBUF]; p += NBUF
        rv = refs[p:p + NBUF]; p += NBUF
        acc_sh = refs[p]; p += 1
        semi = refs[p:p + NBUF]; p += NBUF
        semr = refs[p:p + NBUF]; p += NBUF
        if with_deg:
            deg_hbm = refs[1]
            ones_v, zdeg_v, deg_sh = refs[p:p + 3]
        c = lax.axis_index("c")
        s = lax.axis_index("s")
        n_chunks = jnp.where(c == 0, n0, n1)
        base = jnp.where(c == 0, s * (n0 * K),
                         NS * (n0 * K) + s * (n1 * K))

        zero16 = jnp.zeros((16,), jnp.float32)

        def fill_zrow(i, _):
            r = i // (D // 16)
            col = (i % (D // 16)) * 16
            rv[0][r, pl.ds(col, 16)] = zero16
            return 0
        lax.fori_loop(0, K * (D // 16), fill_zrow, 0)

        if with_deg:
            one16 = jnp.ones((16,), jnp.float32)
            def fill_deg_bufs(i, _):
                zdeg_v[pl.ds(i * 16, 16)] = zero16
                ones_v[pl.ds(i * 16, 16)] = one16
                return 0
            lax.fori_loop(0, K // 16, fill_deg_bufs, 0)

        row0 = s * ROWS_PER_TILE
        def zero_acc(i, _):
            pltpu.sync_copy(rv[0], acc_sh.at[pl.ds(row0 + i * K, K)])
            if with_deg:
                pltpu.sync_copy(zdeg_v,
                                deg_sh.at[pl.ds(row0 + i * K, K)])
            return 0
        lax.fori_loop(0, ROWS_PER_TILE // K, zero_acc, 0)

        plsc.subcore_barrier()

        def idx_load(g, j):
            pltpu.async_copy(src_hbm.at[pl.ds(base + g * K, K)],
                             sv[j], semi[j])
            pltpu.async_copy(dst_hbm.at[pl.ds(base + g * K, K)],
                             dv[j], semi[j])

        def idx_wait(g, j):
            pltpu.make_async_copy(
                src_hbm.at[pl.ds(base + g * K, K)], sv[j], semi[j]).wait()
            pltpu.make_async_copy(
                dst_hbm.at[pl.ds(base + g * K, K)], dv[j], semi[j]).wait()

        def g_start(j):
            pltpu.async_copy(x_hbm.at[sv[j]], rv[j], semr[j])

        def g_wait(j):
            pltpu.make_async_copy(x_hbm.at[sv[j]], rv[j], semr[j]).wait()

        def scat(j):
            pltpu.sync_copy(rv[j], acc_sh.at[dv[j]], add=True)
            if with_deg:
                pltpu.sync_copy(ones_v, deg_sh.at[dv[j]], add=True)

        for j in range(NBUF):
            idx_load(j, j)
        for j in range(NBUF - 1):
            idx_wait(j, j)
            g_start(j)

        def body(t, _):
            g0 = t * NBUF
            for j in range(NBUF):
                g = g0 + j
                jp = (j + NBUF - 1) % NBUF
                idx_wait(g + NBUF - 1, jp)
                g_start(jp)
                g_wait(j)
                scat(j)
                idx_load(g + NBUF, j)
            return 0
        lax.fori_loop(0, n_chunks // NBUF - 1, body, 0)

        idx_wait(n_chunks - 1, NBUF - 1)
        g_start(NBUF - 1)
        for j in range(NBUF):
            g_wait(j)
            scat(j)

        plsc.subcore_barrier()

        pltpu.sync_copy(acc_sh.at[pl.ds(row0, ROWS_PER_TILE)],
                        acc_hbm.at[c, pl.ds(row0, ROWS_PER_TILE)])
        if with_deg:
            pltpu.sync_copy(deg_sh.at[pl.ds(row0, ROWS_PER_TILE)],
                            deg_hbm.at[c, pl.ds(row0, ROWS_PER_TILE)])

    res = agg(x_pad, src, dst)
    if not isinstance(res, (list, tuple)):
        res = (res,)
    return tuple(res)


BLK = 1000


def _tc_layer(acc, deg, x, wn, ws, b2d, relu):
    def body(acc_ref, deg_ref, x_ref, wn_ref, ws_ref, b_ref, o_ref):
        a = acc_ref[0] + acc_ref[1]
        dg = deg_ref[0] + deg_ref[1]
        mean = a / jnp.maximum(dg, 1.0)
        out = jnp.dot(mean, wn_ref[...], preferred_element_type=jnp.float32)
        out = out + jnp.dot(x_ref[...], ws_ref[...],
                            preferred_element_type=jnp.float32)
        out = out + b_ref[...]
        if relu:
            out = jnp.maximum(out, 0.0)
        o_ref[...] = out

    return pl.pallas_call(
        body,
        grid=(N_NODES // BLK,),
        in_specs=[
            pl.BlockSpec((NC, BLK, D), lambda i: (0, i, 0)),
            pl.BlockSpec((NC, BLK, 1), lambda i: (0, i, 0)),
            pl.BlockSpec((BLK, D), lambda i: (i, 0)),
            pl.BlockSpec((D, D), lambda i: (0, 0)),
            pl.BlockSpec((D, D), lambda i: (0, 0)),
            pl.BlockSpec((1, D), lambda i: (0, 0)),
        ],
        out_specs=pl.BlockSpec((BLK, D), lambda i: (i, 0)),
        out_shape=jax.ShapeDtypeStruct((N_NODES, D), jnp.float32),
    )(acc, deg, x, wn, ws, b2d)


def kernel(x, edge_index, W_neigh1, W_self1, b1, W_neigh2, W_self2, b2):
    src = edge_index[0].astype(jnp.int32)
    dst = edge_index[1].astype(jnp.int32)
    e = src.shape[0]
    tp = 2 * (((e + NBUF * NW * K - 1) // (NBUF * NW * K)) * NBUF)
    n0 = max(2 * NBUF, int(round(tp * SLOW_FRAC / 2)) * 2)
    n1 = tp - n0
    e_pad = NS * K * (n0 + n1)
    if e_pad > e:
        src = jnp.concatenate([src, jnp.zeros((e_pad - e,), jnp.int32)])
        dst = jnp.concatenate([dst, jnp.full((e_pad - e,), N_NODES, jnp.int32)])

    acc1, deg = _sc_aggregate(x, src, dst, n0, n1, with_deg=True)
    deg = deg.reshape(NC, N_PAD, 1)
    h = _tc_layer(acc1, deg, x, W_neigh1, W_self1,
                  b1.reshape(1, D), relu=True)
    (acc2,) = _sc_aggregate(h, src, dst, n0, n1, with_deg=False)
    return _tc_layer(acc2, deg, h, W_neigh2, W_self2,
                     b2.reshape(1, D), relu=False)
